# baseline (speedup 1.0000x reference)
import jax
from jax.experimental import pallas as pl
from jax.experimental.pallas import tpu as pltpu
from jax.experimental.pallas import tpu_sc as plsc


def _copy_body(table_hbm, out_hbm):
    pltpu.sync_copy(table_hbm, out_hbm)


def kernel(x, a, onehot_encoding, parameters_encoding_matrix):
    mesh = plsc.ScalarSubcoreMesh(axis_name="c", num_cores=1)
    run = pl.kernel(
        _copy_body,
        out_type=jax.ShapeDtypeStruct(onehot_encoding.shape, onehot_encoding.dtype),
        mesh=mesh,
    )
    return run(onehot_encoding)
